# baseline (speedup 1.0000x reference)
import functools

import jax
import jax.numpy as jnp
from jax import lax
from jax.experimental import pallas as pl
from jax.experimental.pallas import tpu as pltpu
from jax.experimental.pallas import tpu_sc as plsc

_NC = 2
_NS = 16
_R = 32


def _sc_add_kernel(T, D, x_hbm, pos_hbm, out_hbm, xb0, xb1, posbuf,
                   in_s0, in_s1, out_s0, out_s1):
    B = x_hbm.shape[0]
    nw = _NC * _NS
    strip = T // nw
    wid = lax.axis_index("s") * _NC + lax.axis_index("c")
    e0 = wid * strip * D
    chunk = _R * D
    nslice = chunk // 16
    nchunk = strip // _R
    xbufs = (xb0, xb1)
    in_sems = (in_s0, in_s1)
    out_sems = (out_s0, out_s1)

    units = [(c, b) for c in range(nchunk) for b in range(B)]
    n_units = len(units)

    def start_in(u):
        c, b = units[u]
        k = u % 2
        return pltpu.async_copy(
            x_hbm.at[b, pl.ds(e0 + c * chunk, chunk)], xbufs[k], in_sems[k])

    def start_out(u):
        c, b = units[u]
        k = u % 2
        return pltpu.async_copy(
            xbufs[k], out_hbm.at[b, pl.ds(e0 + c * chunk, chunk)], out_sems[k])

    def compute(u):
        k = u % 2
        buf = xbufs[k]

        @plsc.parallel_loop(0, nslice, 1, unroll=16)
        def slice_body(i):
            v = posbuf[pl.ds(i * 16, 16)]
            plsc.addupdate(buf.at[pl.ds(i * 16, 16)], v)

    in_dma = [None] * n_units
    out_dma = [None] * n_units
    in_dma[0] = start_in(0)
    for u in range(n_units):
        c, b = units[u]
        if b == 0:
            pltpu.sync_copy(pos_hbm.at[pl.ds(e0 + c * chunk, chunk)], posbuf)
        if u + 1 < n_units:
            if u - 1 >= 0:
                out_dma[u - 1].wait()
            in_dma[u + 1] = start_in(u + 1)
        in_dma[u].wait()
        compute(u)
        out_dma[u] = start_out(u)
    out_dma[n_units - 1].wait()
    if n_units >= 2:
        out_dma[n_units - 2].wait()


def kernel(x, pos_embedding):
    B, T, D = x.shape
    mesh = plsc.VectorSubcoreMesh(core_axis_name="c", subcore_axis_name="s")
    f = functools.partial(
        pl.kernel,
        mesh=mesh,
        out_type=jax.ShapeDtypeStruct((B, T * D), x.dtype),
        scratch_types=[
            pltpu.VMEM((_R * D,), jnp.float32),
            pltpu.VMEM((_R * D,), jnp.float32),
            pltpu.VMEM((_R * D,), jnp.float32),
            pltpu.SemaphoreType.DMA,
            pltpu.SemaphoreType.DMA,
            pltpu.SemaphoreType.DMA,
            pltpu.SemaphoreType.DMA,
        ],
    )(functools.partial(_sc_add_kernel, T, D))
    out = f(x.reshape(B, T * D), pos_embedding.reshape(-1))
    return out.reshape(B, T, D)


# speedup vs baseline: 2.3747x; 2.3747x over previous
import functools

import jax
import jax.numpy as jnp
from jax import lax
from jax.experimental import pallas as pl
from jax.experimental.pallas import tpu as pltpu
from jax.experimental.pallas import tpu_sc as plsc

_NC = 2
_NS = 16
_R = 32


def _sc_add_kernel(T, x_hbm, pos_hbm, out_hbm, xb0, xb1, posbuf,
                   in_s0, in_s1, out_s0, out_s1):
    B = x_hbm.shape[0]
    D = x_hbm.shape[2]
    nw = _NC * _NS
    strip = T // nw
    wid = lax.axis_index("s") * _NC + lax.axis_index("c")
    t0 = wid * strip
    nslice = D // 16
    nchunk = strip // _R
    xbufs = (xb0, xb1)
    in_sems = (in_s0, in_s1)
    out_sems = (out_s0, out_s1)

    units = [(c, b) for c in range(nchunk) for b in range(B)]
    n_units = len(units)

    def start_in(u):
        c, b = units[u]
        k = u % 2
        return pltpu.async_copy(
            x_hbm.at[b, pl.ds(t0 + c * _R, _R), :], xbufs[k], in_sems[k])

    def start_out(u):
        c, b = units[u]
        k = u % 2
        return pltpu.async_copy(
            xbufs[k], out_hbm.at[b, pl.ds(t0 + c * _R, _R), :], out_sems[k])

    def compute(u):
        k = u % 2
        buf = xbufs[k]

        nsub = 4
        per = nslice // nsub

        @plsc.parallel_loop(0, _R * nsub, 1, unroll=1)
        def qrow_body(i):
            r = i // nsub
            col0 = (i % nsub) * (per * 16)
            for j in range(per):
                v = posbuf[r, pl.ds(col0 + j * 16, 16)]
                plsc.addupdate(buf.at[r, pl.ds(col0 + j * 16, 16)], v)

    in_dma = [None] * n_units
    out_dma = [None] * n_units
    in_dma[0] = start_in(0)
    for u in range(n_units):
        c, b = units[u]
        if b == 0:
            pltpu.sync_copy(pos_hbm.at[pl.ds(t0 + c * _R, _R), :], posbuf)
        if u + 1 < n_units:
            if u - 1 >= 0:
                out_dma[u - 1].wait()
            in_dma[u + 1] = start_in(u + 1)
        in_dma[u].wait()
        compute(u)
        out_dma[u] = start_out(u)
    out_dma[n_units - 1].wait()
    if n_units >= 2:
        out_dma[n_units - 2].wait()


def kernel(x, pos_embedding):
    B, T, D = x.shape
    mesh = plsc.VectorSubcoreMesh(core_axis_name="c", subcore_axis_name="s")
    f = functools.partial(
        pl.kernel,
        mesh=mesh,
        out_type=jax.ShapeDtypeStruct((B, T, D), x.dtype),
        scratch_types=[
            pltpu.VMEM((_R, D), jnp.float32),
            pltpu.VMEM((_R, D), jnp.float32),
            pltpu.VMEM((_R, D), jnp.float32),
            pltpu.SemaphoreType.DMA,
            pltpu.SemaphoreType.DMA,
            pltpu.SemaphoreType.DMA,
            pltpu.SemaphoreType.DMA,
        ],
    )(functools.partial(_sc_add_kernel, T))
    return f(x, pos_embedding)


# speedup vs baseline: 4.6020x; 1.9379x over previous
import jax
import jax.numpy as jnp
from jax.experimental import pallas as pl


_RT = 2048


def _add_pos_kernel(x_ref, pos_ref, out_ref):
    out_ref[0, :, :] = x_ref[0, :, :] + pos_ref[...]


def kernel(x, pos_embedding):
    B, T, D = x.shape
    rt = _RT if T % _RT == 0 else T
    grid = (T // rt, B)
    return pl.pallas_call(
        _add_pos_kernel,
        grid=grid,
        in_specs=[
            pl.BlockSpec((1, rt, D), lambda i, b: (b, i, 0)),
            pl.BlockSpec((rt, D), lambda i, b: (i, 0)),
        ],
        out_specs=pl.BlockSpec((1, rt, D), lambda i, b: (b, i, 0)),
        out_shape=jax.ShapeDtypeStruct((B, T, D), x.dtype),
    )(x, pos_embedding)
